# baseline (speedup 1.0000x reference)
import functools

import jax
import jax.numpy as jnp
from jax import lax
from jax.experimental import pallas as pl
from jax.experimental.pallas import tpu as pltpu
from jax.experimental.pallas import tpu_sc as plsc

NUM_DAYS = 512
EMB = 128
NC, NS = 2, 16
NW = NC * NS
CHUNK = 128
CNT_W = 128
NSLOT = 3


def _sc_partial_sums(ids_packed, news_table, cat_table, zeros_z, ones_c):
    n_chunks_total = ids_packed.shape[0]
    n_chunks = n_chunks_total // NW
    rows_per_sub = NUM_DAYS // NS
    cat_rows = cat_table.shape[0]
    mesh = plsc.VectorSubcoreMesh(core_axis_name="c", subcore_axis_name="s")

    step = -(-cat_rows // NS)
    step += (-step) % 8
    cat_starts, cat_sizes = [], []
    for sid_py in range(NS):
        start = sid_py * step
        sz = max(0, min(step, cat_rows - start))
        cat_starts.append(start)
        cat_sizes.append(sz)

    @functools.partial(
        pl.kernel,
        out_type=(
            jax.ShapeDtypeStruct((NC, NUM_DAYS, EMB), jnp.float32),
            jax.ShapeDtypeStruct((NC, NUM_DAYS, CNT_W), jnp.float32),
        ),
        mesh=mesh,
        scratch_types=[
            pltpu.VMEM((n_chunks, 3, CHUNK), jnp.int32),
            pltpu.VMEM((NSLOT, CHUNK, EMB), jnp.float32),
            pltpu.VMEM((NSLOT, CHUNK, EMB), jnp.float32),
            pltpu.VMEM((CHUNK, CNT_W), jnp.float32),
            pltpu.VMEM_SHARED((NUM_DAYS, EMB), jnp.float32),
            pltpu.VMEM_SHARED((NUM_DAYS, CNT_W), jnp.float32),
        ] + [pltpu.SemaphoreType.DMA] * (2 * NSLOT),
    )
    def k(ids_hbm, news_hbm, cat_hbm, z0_hbm, ones_hbm, zp_hbm, cp_hbm,
          idx_v, bufn_v, bufc_v, ones_v, zacc_s, cacc_s, *sems):
        gsem = sems[:NSLOT]
        ssem = sems[NSLOT:]
        core = lax.axis_index("c")
        sid = lax.axis_index("s")
        wid = sid * NC + core
        my_rows = pl.ds(sid * rows_per_sub, rows_per_sub)

        pltpu.sync_copy(z0_hbm.at[my_rows], zacc_s.at[my_rows])
        pltpu.sync_copy(z0_hbm.at[my_rows], cacc_s.at[my_rows])
        pltpu.sync_copy(ones_hbm, ones_v)
        pltpu.sync_copy(ids_hbm.at[pl.ds(wid * n_chunks, n_chunks)], idx_v)
        plsc.subcore_barrier()

        gd = [None] * NSLOT
        sd = [None] * NSLOT

        def fire_scatters(i):
            slot = i % NSLOT
            for d in gd[slot]:
                d.wait()
            day_idx = idx_v.at[i].at[2]
            sd[slot] = [
                pltpu.async_copy(bufn_v.at[slot], zacc_s.at[day_idx],
                                 ssem[slot], add=True),
                pltpu.async_copy(bufc_v.at[slot], zacc_s.at[day_idx],
                                 ssem[slot], add=True),
                pltpu.async_copy(ones_v, cacc_s.at[day_idx],
                                 ssem[slot], add=True),
            ]

        for i in range(n_chunks):
            s = i % NSLOT
            if sd[s] is not None:
                for d in sd[s]:
                    d.wait()
                sd[s] = None
            gd[s] = [
                pltpu.async_copy(news_hbm.at[idx_v.at[i].at[0]],
                                 bufn_v.at[s], gsem[s]),
                pltpu.async_copy(cat_hbm.at[idx_v.at[i].at[1]],
                                 bufc_v.at[s], gsem[s]),
            ]
            if i >= 1:
                fire_scatters(i - 1)
        fire_scatters(n_chunks - 1)
        for slot in range(NSLOT):
            if sd[slot] is not None:
                for d in sd[slot]:
                    d.wait()

        plsc.subcore_barrier()
        pltpu.sync_copy(zacc_s.at[my_rows], zp_hbm.at[core].at[my_rows])
        pltpu.sync_copy(cacc_s.at[my_rows], cp_hbm.at[core].at[my_rows])

    return k(ids_packed, news_table, cat_table, zeros_z, ones_c)


def _tc_combine(zp, cp):
    def body(zp_ref, cp_ref, out_ref):
        z = zp_ref[0] + zp_ref[1]
        c = cp_ref[0, :, 0:1] + cp_ref[1, :, 0:1]
        out_ref[...] = z / jnp.maximum(c, 1.0)

    return pl.pallas_call(
        body,
        out_shape=jax.ShapeDtypeStruct((NUM_DAYS, EMB), jnp.float32),
    )(zp, cp)


def kernel(news_ids, category_ids, day_ids, delta_days, news_table, cat_table):
    n = news_ids.shape[0]
    n_chunks_total = n // CHUNK
    ids_packed = jnp.stack(
        [news_ids.astype(jnp.int32).reshape(n_chunks_total, CHUNK),
         category_ids.astype(jnp.int32).reshape(n_chunks_total, CHUNK),
         day_ids.astype(jnp.int32).reshape(n_chunks_total, CHUNK)],
        axis=1)
    zeros_z = jnp.zeros((NUM_DAYS, EMB), jnp.float32)
    ones_c = jnp.ones((CHUNK, CNT_W), jnp.float32)
    zp, cp = _sc_partial_sums(ids_packed, news_table, cat_table, zeros_z,
                              ones_c)
    Z = _tc_combine(zp, cp)
    return (Z, delta_days.astype(jnp.float32))


# speedup vs baseline: 1.1413x; 1.1413x over previous
import functools

import jax
import jax.numpy as jnp
from jax import lax
from jax.experimental import pallas as pl
from jax.experimental.pallas import tpu as pltpu
from jax.experimental.pallas import tpu_sc as plsc

NUM_DAYS = 512
EMB = 128
NC, NS = 2, 16
NW = NC * NS
CHUNK = 128
CNT_W = 128
NSLOT = 3
LANES = 16


def _sc_partial_sums(nid2, cid2, did2, news_table, cat_table):
    n_chunks_total = nid2.shape[0]
    n_chunks = n_chunks_total // NW
    rows_per_sub = NUM_DAYS // NS
    mesh = plsc.VectorSubcoreMesh(core_axis_name="c", subcore_axis_name="s")

    @functools.partial(
        pl.kernel,
        out_type=(
            jax.ShapeDtypeStruct((NC, NUM_DAYS, EMB), jnp.float32),
            jax.ShapeDtypeStruct((NC, NUM_DAYS, CNT_W), jnp.float32),
        ),
        mesh=mesh,
        scratch_types=[
            pltpu.VMEM((3, n_chunks, CHUNK), jnp.int32),
            pltpu.VMEM((NSLOT, CHUNK, EMB), jnp.float32),
            pltpu.VMEM((NSLOT, CHUNK, EMB), jnp.float32),
            pltpu.VMEM((CHUNK, CNT_W), jnp.float32),
            pltpu.VMEM((NUM_DAYS // NS, EMB), jnp.float32),
            pltpu.VMEM_SHARED((NUM_DAYS, EMB), jnp.float32),
            pltpu.VMEM_SHARED((NUM_DAYS, CNT_W), jnp.float32),
        ] + [pltpu.SemaphoreType.DMA] * (2 * NSLOT),
    )
    def k(nid_hbm, cid_hbm, did_hbm, news_hbm, cat_hbm, zp_hbm, cp_hbm,
          idx_v, bufn_v, bufc_v, ones_v, zinit_v, zacc_s, cacc_s, *sems):
        gsem = sems[:NSLOT]
        ssem = sems[NSLOT:]
        core = lax.axis_index("c")
        sid = lax.axis_index("s")
        wid = sid * NC + core
        my_rows = pl.ds(sid * rows_per_sub, rows_per_sub)
        my_chunks = pl.ds(wid * n_chunks, n_chunks)

        pltpu.sync_copy(nid_hbm.at[my_chunks], idx_v.at[0])
        pltpu.sync_copy(cid_hbm.at[my_chunks], idx_v.at[1])
        pltpu.sync_copy(did_hbm.at[my_chunks], idx_v.at[2])

        gd = [None] * NSLOT
        sd = [None] * NSLOT

        def fire_gathers(i):
            s = i % NSLOT
            gd[s] = [
                pltpu.async_copy(news_hbm.at[idx_v.at[0].at[i]],
                                 bufn_v.at[s], gsem[s]),
                pltpu.async_copy(cat_hbm.at[idx_v.at[1].at[i]],
                                 bufc_v.at[s], gsem[s]),
            ]

        def fire_scatters(i):
            s = i % NSLOT
            for d in gd[s]:
                d.wait()
            day_idx = idx_v.at[2].at[i]
            sd[s] = [
                pltpu.async_copy(bufn_v.at[s], zacc_s.at[day_idx],
                                 ssem[s], add=True),
                pltpu.async_copy(bufc_v.at[s], zacc_s.at[day_idx],
                                 ssem[s], add=True),
                pltpu.async_copy(ones_v, cacc_s.at[day_idx],
                                 ssem[s], add=True),
            ]

        fire_gathers(0)

        zero_r = jnp.zeros((LANES,), jnp.float32)
        one_r = jnp.full((LANES,), 1.0, jnp.float32)

        @pl.loop(0, rows_per_sub)
        def _(r):
            for c in range(EMB // LANES):
                zinit_v[r, pl.ds(c * LANES, LANES)] = zero_r

        pltpu.sync_copy(zinit_v, zacc_s.at[my_rows])
        pltpu.sync_copy(zinit_v, cacc_s.at[my_rows])

        @pl.loop(0, CHUNK)
        def _(r):
            for c in range(CNT_W // LANES):
                ones_v[r, pl.ds(c * LANES, LANES)] = one_r

        plsc.subcore_barrier()

        for i in range(1, n_chunks):
            s = i % NSLOT
            if sd[s] is not None:
                for d in sd[s]:
                    d.wait()
                sd[s] = None
            fire_gathers(i)
            fire_scatters(i - 1)
        fire_scatters(n_chunks - 1)
        for slot in range(NSLOT):
            if sd[slot] is not None:
                for d in sd[slot]:
                    d.wait()

        plsc.subcore_barrier()
        pltpu.sync_copy(zacc_s.at[my_rows], zp_hbm.at[core].at[my_rows])
        pltpu.sync_copy(cacc_s.at[my_rows], cp_hbm.at[core].at[my_rows])

    return k(nid2, cid2, did2, news_table, cat_table)


def _tc_combine(zp, cp):
    def body(zp_ref, cp_ref, out_ref):
        z = zp_ref[0] + zp_ref[1]
        c = cp_ref[0, :, 0:1] + cp_ref[1, :, 0:1]
        out_ref[...] = z / jnp.maximum(c, 1.0)

    return pl.pallas_call(
        body,
        out_shape=jax.ShapeDtypeStruct((NUM_DAYS, EMB), jnp.float32),
    )(zp, cp)


def kernel(news_ids, category_ids, day_ids, delta_days, news_table, cat_table):
    n = news_ids.shape[0]
    n_chunks_total = n // CHUNK
    nid2 = news_ids.astype(jnp.int32).reshape(n_chunks_total, CHUNK)
    cid2 = category_ids.astype(jnp.int32).reshape(n_chunks_total, CHUNK)
    did2 = day_ids.astype(jnp.int32).reshape(n_chunks_total, CHUNK)
    zp, cp = _sc_partial_sums(nid2, cid2, did2, news_table, cat_table)
    Z = _tc_combine(zp, cp)
    return (Z, delta_days.astype(jnp.float32))


# speedup vs baseline: 1.2664x; 1.1096x over previous
import functools

import jax
import jax.numpy as jnp
from jax import lax
from jax.experimental import pallas as pl
from jax.experimental.pallas import tpu as pltpu
from jax.experimental.pallas import tpu_sc as plsc

NUM_DAYS = 512
EMB = 128
NC, NS = 2, 16
NW = NC * NS
CHUNK = 128
CNT_W = 128
NSLOT = 3
LANES = 16


def _sc_partial_sums(nid2, cid2, did2, news_table, cat_table):
    n_chunks_total = nid2.shape[0]
    n_chunks = n_chunks_total // NW
    rows_per_sub = NUM_DAYS // NS
    mesh = plsc.VectorSubcoreMesh(core_axis_name="c", subcore_axis_name="s")

    @functools.partial(
        pl.kernel,
        out_type=jax.ShapeDtypeStruct((NC, NUM_DAYS, EMB), jnp.float32),
        mesh=mesh,
        scratch_types=[
            pltpu.VMEM((3, n_chunks, CHUNK), jnp.int32),
            pltpu.VMEM((NSLOT, CHUNK, EMB), jnp.float32),
            pltpu.VMEM((NSLOT, CHUNK, EMB), jnp.float32),
            pltpu.VMEM((NUM_DAYS // NS, EMB), jnp.float32),
            pltpu.VMEM_SHARED((NUM_DAYS, EMB), jnp.float32),
        ] + [pltpu.SemaphoreType.DMA] * (2 * NSLOT),
    )
    def k(nid_hbm, cid_hbm, did_hbm, news_hbm, cat_hbm, zp_hbm,
          idx_v, bufn_v, bufc_v, zinit_v, zacc_s, *sems):
        gsem = sems[:NSLOT]
        ssem = sems[NSLOT:]
        core = lax.axis_index("c")
        sid = lax.axis_index("s")
        wid = sid * NC + core
        my_rows = pl.ds(sid * rows_per_sub, rows_per_sub)
        my_chunks = pl.ds(wid * n_chunks, n_chunks)

        pltpu.sync_copy(nid_hbm.at[my_chunks], idx_v.at[0])
        pltpu.sync_copy(cid_hbm.at[my_chunks], idx_v.at[1])
        pltpu.sync_copy(did_hbm.at[my_chunks], idx_v.at[2])

        gd = [None] * NSLOT
        sd = [None] * NSLOT

        def fire_gathers(i):
            s = i % NSLOT
            gd[s] = [
                pltpu.async_copy(news_hbm.at[idx_v.at[0].at[i]],
                                 bufn_v.at[s], gsem[s]),
                pltpu.async_copy(cat_hbm.at[idx_v.at[1].at[i]],
                                 bufc_v.at[s], gsem[s]),
            ]

        def fire_scatters(i):
            s = i % NSLOT
            for d in gd[s]:
                d.wait()
            day_idx = idx_v.at[2].at[i]
            sd[s] = [
                pltpu.async_copy(bufn_v.at[s], zacc_s.at[day_idx],
                                 ssem[s], add=True),
                pltpu.async_copy(bufc_v.at[s], zacc_s.at[day_idx],
                                 ssem[s], add=True),
            ]

        fire_gathers(0)

        zero_r = jnp.zeros((LANES,), jnp.float32)

        @pl.loop(0, rows_per_sub)
        def _(r):
            for c in range(EMB // LANES):
                zinit_v[r, pl.ds(c * LANES, LANES)] = zero_r

        pltpu.sync_copy(zinit_v, zacc_s.at[my_rows])

        plsc.subcore_barrier()

        for i in range(1, n_chunks):
            s = i % NSLOT
            if sd[s] is not None:
                for d in sd[s]:
                    d.wait()
                sd[s] = None
            fire_gathers(i)
            fire_scatters(i - 1)
        fire_scatters(n_chunks - 1)
        for slot in range(NSLOT):
            if sd[slot] is not None:
                for d in sd[slot]:
                    d.wait()

        plsc.subcore_barrier()
        pltpu.sync_copy(zacc_s.at[my_rows], zp_hbm.at[core].at[my_rows])

    return k(nid2, cid2, did2, news_table, cat_table)


HB = 2048


def _tc_day_histogram(did2):
    n = did2.size

    def body(ids_ref, out_ref):
        @pl.when(pl.program_id(0) == 0)
        def _():
            out_ref[...] = jnp.zeros_like(out_ref)

        x = ids_ref[...].reshape(HB, 1)
        days = lax.broadcasted_iota(jnp.int32, (1, NUM_DAYS), 1)
        eq = (x == days).astype(jnp.float32)
        out_ref[...] += jnp.sum(eq, axis=0, keepdims=True)

    return pl.pallas_call(
        body,
        grid=(n // HB,),
        in_specs=[pl.BlockSpec((1, 1, HB), lambda i: (i, 0, 0))],
        out_specs=pl.BlockSpec((1, NUM_DAYS), lambda i: (0, 0)),
        out_shape=jax.ShapeDtypeStruct((1, NUM_DAYS), jnp.float32),
    )(did2.reshape(n // HB, 1, HB))


def _tc_combine(zp, counts):
    def body(zp_ref, cnt_ref, out_ref):
        z = zp_ref[0] + zp_ref[1]
        c = cnt_ref[...].reshape(NUM_DAYS, 1)
        out_ref[...] = z / jnp.maximum(c, 1.0)

    return pl.pallas_call(
        body,
        out_shape=jax.ShapeDtypeStruct((NUM_DAYS, EMB), jnp.float32),
    )(zp, counts)


def kernel(news_ids, category_ids, day_ids, delta_days, news_table, cat_table):
    n = news_ids.shape[0]
    n_chunks_total = n // CHUNK
    nid2 = news_ids.astype(jnp.int32).reshape(n_chunks_total, CHUNK)
    cid2 = category_ids.astype(jnp.int32).reshape(n_chunks_total, CHUNK)
    did2 = day_ids.astype(jnp.int32).reshape(n_chunks_total, CHUNK)
    counts = _tc_day_histogram(did2)
    zp = _sc_partial_sums(nid2, cid2, did2, news_table, cat_table)
    Z = _tc_combine(zp, counts)
    return (Z, delta_days.astype(jnp.float32))


# speedup vs baseline: 1.2908x; 1.0193x over previous
import functools

import jax
import jax.numpy as jnp
from jax import lax
from jax.experimental import pallas as pl
from jax.experimental.pallas import tpu as pltpu
from jax.experimental.pallas import tpu_sc as plsc

NUM_DAYS = 512
EMB = 128
NC, NS = 2, 16
NW = NC * NS
CHUNK = 128
CNT_W = 128
NSLOT = 3
LANES = 16


def _sc_partial_sums(nid2, cid2, did2, news_table, cat_table):
    n_chunks_total = nid2.shape[0]
    n_chunks = n_chunks_total // NW
    rows_per_sub = NUM_DAYS // NS
    mesh = plsc.VectorSubcoreMesh(core_axis_name="c", subcore_axis_name="s")

    @functools.partial(
        pl.kernel,
        out_type=jax.ShapeDtypeStruct((NC, NUM_DAYS, EMB), jnp.float32),
        mesh=mesh,
        scratch_types=[
            pltpu.VMEM((3, n_chunks, CHUNK), jnp.int32),
            pltpu.VMEM((NSLOT, CHUNK, EMB), jnp.float32),
            pltpu.VMEM((NSLOT, CHUNK, EMB), jnp.float32),
            pltpu.VMEM((NUM_DAYS // NS, EMB), jnp.float32),
            pltpu.VMEM_SHARED((NUM_DAYS, EMB), jnp.float32),
        ] + [pltpu.SemaphoreType.DMA] * (2 * NSLOT),
    )
    def k(nid_hbm, cid_hbm, did_hbm, news_hbm, cat_hbm, zp_hbm,
          idx_v, bufn_v, bufc_v, zinit_v, zacc_s, *sems):
        gsem = sems[:NSLOT]
        ssem = sems[NSLOT:]
        core = lax.axis_index("c")
        sid = lax.axis_index("s")
        wid = sid * NC + core
        my_rows = pl.ds(sid * rows_per_sub, rows_per_sub)
        my_chunks = pl.ds(wid * n_chunks, n_chunks)

        pltpu.sync_copy(nid_hbm.at[my_chunks], idx_v.at[0])
        pltpu.sync_copy(cid_hbm.at[my_chunks], idx_v.at[1])
        pltpu.sync_copy(did_hbm.at[my_chunks], idx_v.at[2])

        gd = [None] * NSLOT
        sd = [None] * NSLOT

        def fire_gathers(i):
            s = i % NSLOT
            gd[s] = [
                pltpu.async_copy(news_hbm.at[idx_v.at[0].at[i]],
                                 bufn_v.at[s], gsem[s]),
                pltpu.async_copy(cat_hbm.at[idx_v.at[1].at[i]],
                                 bufc_v.at[s], gsem[s]),
            ]

        def fire_scatters(i):
            s = i % NSLOT
            for d in gd[s]:
                d.wait()

            @pl.loop(0, CHUNK)
            def _(r):
                for c in range(EMB // LANES):
                    sl = pl.ds(c * LANES, LANES)
                    bufn_v[s, r, sl] = bufn_v[s, r, sl] + bufc_v[s, r, sl]

            day_idx = idx_v.at[2].at[i]
            sd[s] = [
                pltpu.async_copy(bufn_v.at[s], zacc_s.at[day_idx],
                                 ssem[s], add=True),
            ]

        fire_gathers(0)

        zero_r = jnp.zeros((LANES,), jnp.float32)

        @pl.loop(0, rows_per_sub)
        def _(r):
            for c in range(EMB // LANES):
                zinit_v[r, pl.ds(c * LANES, LANES)] = zero_r

        pltpu.sync_copy(zinit_v, zacc_s.at[my_rows])

        plsc.subcore_barrier()

        for i in range(1, n_chunks):
            s = i % NSLOT
            if sd[s] is not None:
                for d in sd[s]:
                    d.wait()
                sd[s] = None
            fire_gathers(i)
            fire_scatters(i - 1)
        fire_scatters(n_chunks - 1)
        for slot in range(NSLOT):
            if sd[slot] is not None:
                for d in sd[slot]:
                    d.wait()

        plsc.subcore_barrier()
        pltpu.sync_copy(zacc_s.at[my_rows], zp_hbm.at[core].at[my_rows])

    return k(nid2, cid2, did2, news_table, cat_table)


HB = 2048


def _tc_day_histogram(did2):
    n = did2.size

    def body(ids_ref, out_ref):
        @pl.when(pl.program_id(0) == 0)
        def _():
            out_ref[...] = jnp.zeros_like(out_ref)

        x = ids_ref[...].reshape(HB, 1)
        days = lax.broadcasted_iota(jnp.int32, (1, NUM_DAYS), 1)
        eq = (x == days).astype(jnp.float32)
        out_ref[...] += jnp.sum(eq, axis=0, keepdims=True)

    return pl.pallas_call(
        body,
        grid=(n // HB,),
        in_specs=[pl.BlockSpec((1, 1, HB), lambda i: (i, 0, 0))],
        out_specs=pl.BlockSpec((1, NUM_DAYS), lambda i: (0, 0)),
        out_shape=jax.ShapeDtypeStruct((1, NUM_DAYS), jnp.float32),
    )(did2.reshape(n // HB, 1, HB))


def _tc_combine(zp, counts):
    def body(zp_ref, cnt_ref, out_ref):
        z = zp_ref[0] + zp_ref[1]
        c = cnt_ref[...].reshape(NUM_DAYS, 1)
        out_ref[...] = z / jnp.maximum(c, 1.0)

    return pl.pallas_call(
        body,
        out_shape=jax.ShapeDtypeStruct((NUM_DAYS, EMB), jnp.float32),
    )(zp, counts)


def kernel(news_ids, category_ids, day_ids, delta_days, news_table, cat_table):
    n = news_ids.shape[0]
    n_chunks_total = n // CHUNK
    nid2 = news_ids.astype(jnp.int32).reshape(n_chunks_total, CHUNK)
    cid2 = category_ids.astype(jnp.int32).reshape(n_chunks_total, CHUNK)
    did2 = day_ids.astype(jnp.int32).reshape(n_chunks_total, CHUNK)
    counts = _tc_day_histogram(did2)
    zp = _sc_partial_sums(nid2, cid2, did2, news_table, cat_table)
    Z = _tc_combine(zp, counts)
    return (Z, delta_days.astype(jnp.float32))
